# Initial kernel scaffold; baseline (speedup 1.0000x reference)
import jax
import jax.numpy as jnp
from jax.experimental import pallas as pl


def kernel(positions, tables):
    raise NotImplementedError("write your pallas kernel here")


# speedup vs baseline: 17.6407x; 17.6407x over previous
import functools
import math

import jax
import jax.numpy as jnp
from jax import lax
from jax.experimental import pallas as pl
from jax.experimental.pallas import tpu as pltpu
from jax.experimental.pallas import tpu_sc as plsc

_NUM_LEVELS = 16
_MIN_RES = 16
_MAX_RES = 2048
_GROWTH = math.exp((math.log(_MAX_RES) - math.log(_MIN_RES)) / (_NUM_LEVELS - 1))
_RES = [int(_MIN_RES * _GROWTH ** i) for i in range(_NUM_LEVELS)]
_MOD = 4096
_Q1 = 2654435761 % _MOD
_Q2 = 805459861 % _MOD

_NC = 2
_NS = 16
_NW = _NC * _NS
_LANES = 16

_N = 524288
_PTS_PER_W = _N // _NW
_BLK = 2048
_NBLK = _PTS_PER_W // _BLK
_NVEC = _BLK // _LANES
_LEVELS_PER_PASS = 8


def _make_kernel():
    mesh = plsc.VectorSubcoreMesh(
        core_axis_name="c", subcore_axis_name="s",
        num_cores=_NC, num_subcores=_NS)

    @functools.partial(
        pl.kernel,
        mesh=mesh,
        out_type=jax.ShapeDtypeStruct((_N, 2 * _NUM_LEVELS), jnp.float32),
        compiler_params=pltpu.CompilerParams(use_tc_tiling_on_sc=False,
                                             needs_layout_passes=False),
        scratch_types=[
            pltpu.VMEM((_LEVELS_PER_PASS * 2 * _MOD,), jnp.float32),
            pltpu.VMEM((_BLK * 3,), jnp.float32),
            pltpu.VMEM((_BLK, 2 * _LEVELS_PER_PASS), jnp.float32),
        ],
    )
    def hash_encode(pos_hbm, hot_hbm, out_hbm, tab_v, pos_v, out_v):

        wid = lax.axis_index("s") * _NC + lax.axis_index("c")
        base = wid * _PTS_PER_W

        iota = lax.iota(jnp.int32, _LANES)
        iota3 = iota * 3

        half = _LEVELS_PER_PASS * 2 * _MOD
        for p in range(2):
            pltpu.sync_copy(hot_hbm.at[pl.ds(half * p, half)], tab_v)

            def blk_body(blk, _, p=p):
                row0 = base + blk * _BLK
                pltpu.sync_copy(pos_hbm.at[pl.ds(row0 * 3, _BLK * 3)], pos_v)

                def vec_body(v, _, p=p):
                    s = v * 48
                    ix = iota3 + s
                    x = plsc.load_gather(pos_v, [ix])
                    y = plsc.load_gather(pos_v, [ix + 1])
                    z = plsc.load_gather(pos_v, [ix + 2])
                    pt = iota + v * _LANES
                    for j in range(_LEVELS_PER_PASS):
                        r = _RES[_LEVELS_PER_PASS * p + j]
                        cx = jnp.minimum((x * jnp.float32(r - 1)).astype(jnp.int32), r - 1)
                        cy = jnp.minimum((y * jnp.float32(r - 1)).astype(jnp.int32), r - 1)
                        cz = jnp.minimum((z * jnp.float32(r - 1)).astype(jnp.int32), r - 1)
                        h = (cx + cy * _Q1 + cz * _Q2) & (_MOD - 1)
                        w = h * 2 + (2 * _MOD * j)
                        g0 = plsc.load_gather(tab_v, [w])
                        g1 = plsc.load_gather(tab_v, [w + 1])
                        plsc.store_scatter(out_v, [pt, jnp.full((_LANES,), 2 * j, jnp.int32)], g0)
                        plsc.store_scatter(out_v, [pt, jnp.full((_LANES,), 2 * j + 1, jnp.int32)], g1)
                    return jnp.int32(0)

                lax.fori_loop(jnp.int32(0), jnp.int32(_NVEC), vec_body,
                              jnp.int32(0))
                pltpu.sync_copy(
                    out_v,
                    out_hbm.at[pl.ds(row0, _BLK),
                               pl.ds(2 * _LEVELS_PER_PASS * p, 2 * _LEVELS_PER_PASS)])
                return jnp.int32(0)

            lax.fori_loop(jnp.int32(0), jnp.int32(_NBLK), blk_body,
                          jnp.int32(0))

    return hash_encode


_KERNEL_CACHE = []


def kernel(positions, tables):
    if not _KERNEL_CACHE:
        _KERNEL_CACHE.append(_make_kernel())
    pos_flat = positions.reshape(-1)
    hot = jnp.stack([t[:_MOD] for t in tables]).reshape(-1)
    return _KERNEL_CACHE[0](pos_flat, hot)


# speedup vs baseline: 68.6886x; 3.8938x over previous
import functools
import math

import jax
import jax.numpy as jnp
from jax import lax
from jax.experimental import pallas as pl
from jax.experimental.pallas import tpu as pltpu
from jax.experimental.pallas import tpu_sc as plsc

_NUM_LEVELS = 16
_MIN_RES = 16
_MAX_RES = 2048
_GROWTH = math.exp((math.log(_MAX_RES) - math.log(_MIN_RES)) / (_NUM_LEVELS - 1))
_RES = [int(_MIN_RES * _GROWTH ** i) for i in range(_NUM_LEVELS)]
_MOD = 4096
_Q1 = 2654435761 % _MOD
_Q2 = 805459861 % _MOD

_NC = 2
_NS = 16
_NW = _NC * _NS
_LANES = 16

_N = 524288
_PTS_PER_W = _N // _NW
_BLK = 1024
_NBLK = _PTS_PER_W // _BLK
_NVEC = _BLK // _LANES
_LPP = 8
_PT_TILES = _N // 128
_BLK_TILES = _BLK // 128


def _make_kernel():
    mesh = plsc.VectorSubcoreMesh(
        core_axis_name="c", subcore_axis_name="s",
        num_cores=_NC, num_subcores=_NS)

    @functools.partial(
        pl.kernel,
        mesh=mesh,
        out_type=jax.ShapeDtypeStruct((4, _PT_TILES, 8, 128), jnp.float32),
        compiler_params=pltpu.CompilerParams(use_tc_tiling_on_sc=False,
                                             needs_layout_passes=False),
        scratch_types=[
            pltpu.VMEM((_LPP * 2 * _MOD,), jnp.float32),
            pltpu.VMEM((3, _BLK), jnp.float32),
            pltpu.VMEM((2, _BLK_TILES, 8, 128), jnp.float32),
        ],
    )
    def hash_encode(pos_hbm, hot_hbm, out_hbm, tab_v, pos_v, out_v):
        wid = lax.axis_index("s") * _NC + lax.axis_index("c")
        base = wid * _PTS_PER_W

        half = _LPP * 2 * _MOD
        for p in range(2):
            pltpu.sync_copy(hot_hbm.at[pl.ds(half * p, half)], tab_v)

            def blk_body(blk, _, p=p):
                row0 = base + blk * _BLK
                for c in range(3):
                    pltpu.sync_copy(pos_hbm.at[jnp.int32(c), pl.ds(row0, _BLK)],
                                    pos_v.at[jnp.int32(c)])

                def vec_body(v, _, p=p):
                    off = v * _LANES
                    x = pos_v[jnp.int32(0), pl.ds(off, _LANES)]
                    y = pos_v[jnp.int32(1), pl.ds(off, _LANES)]
                    z = pos_v[jnp.int32(2), pl.ds(off, _LANES)]
                    pti = lax.div(v, jnp.int32(8))
                    poff = lax.rem(v, jnp.int32(8)) * _LANES
                    for j in range(_LPP):
                        r = _RES[_LPP * p + j]
                        cx = jnp.minimum((x * jnp.float32(r - 1)).astype(jnp.int32), r - 1)
                        cy = jnp.minimum((y * jnp.float32(r - 1)).astype(jnp.int32), r - 1)
                        cz = jnp.minimum((z * jnp.float32(r - 1)).astype(jnp.int32), r - 1)
                        h = (cx + cy * _Q1 + cz * _Q2) & (_MOD - 1)
                        w = h * 2 + (2 * _MOD * j)
                        g0 = plsc.load_gather(tab_v, [w])
                        g1 = plsc.load_gather(tab_v, [w + 1])
                        for fe, g in ((0, g0), (1, g1)):
                            f = 2 * j + fe
                            k = jnp.int32(f // 8)
                            fi = f % 8
                            out_v[k, pti, jnp.int32(fi), pl.ds(poff, _LANES)] = g
                    return jnp.int32(0)

                lax.fori_loop(jnp.int32(0), jnp.int32(_NVEC), vec_body,
                              jnp.int32(0))
                ptile0 = lax.div(row0, jnp.int32(128))
                for k in range(2):
                    pltpu.sync_copy(
                        out_v.at[jnp.int32(k)],
                        out_hbm.at[jnp.int32(2 * p + k),
                                   pl.ds(ptile0, _BLK_TILES)])
                return jnp.int32(0)

            lax.fori_loop(jnp.int32(0), jnp.int32(_NBLK), blk_body,
                          jnp.int32(0))

    return hash_encode


_KERNEL_CACHE = []


def kernel(positions, tables):
    if not _KERNEL_CACHE:
        _KERNEL_CACHE.append(_make_kernel())
    pos_t = positions.T
    hot = jnp.stack([t[:_MOD] for t in tables]).reshape(-1)
    out4d = _KERNEL_CACHE[0](pos_t, hot)
    return out4d.transpose(1, 3, 0, 2).reshape(_N, 2 * _NUM_LEVELS)
